# baseline (speedup 1.0000x reference)
import jax
import jax.numpy as jnp
from jax import lax
from jax.experimental import pallas as pl
from jax.experimental.pallas import tpu as pltpu
from jax.experimental.pallas import tpu_sc as plsc

_B, _S, _D, _E, _TOPK = 4, 8192, 4096, 64, 8
_H = _D // 4
_N = _B * _S
_BLK_M = 1024

_SLICES = 4
_NS = _N // _SLICES
_NWORKERS = 32
_CHUNK = _NS // _NWORKERS
_LANES = 16
_GROUPS = _CHUNK // _LANES

_NEG_INF = float("-inf")


def _tree(op, xs):
    xs = list(xs)
    while len(xs) > 1:
        nxt = [op(xs[i], xs[i + 1]) for i in range(0, len(xs) - 1, 2)]
        if len(xs) % 2:
            nxt.append(xs[-1])
        xs = nxt
    return xs[0]


def _mlp_body(x_ref, w1_ref, b1_ref, w2_ref, b2_ref, t_ref, *rest):
    gate_ref, gate_t_ref = rest[-2], rest[-1]
    x = x_ref[...]
    h = jnp.dot(x, w1_ref[...], preferred_element_type=jnp.float32)
    h = jnp.maximum(h + b1_ref[...], 0.0)
    g = jnp.dot(h, w2_ref[...], preferred_element_type=jnp.float32)
    g = (g + b2_ref[...]) * (1.0 / t_ref[0])
    gate_ref[...] = g
    gate_t_ref[...] = g.T


def _mlp_call_kwargs(si):
    base = si * (_NS // _BLK_M)
    in_specs = [
        pl.BlockSpec((_BLK_M, _D), lambda i: (i + base, 0)),
        pl.BlockSpec((_D, _H), lambda i: (0, 0)),
        pl.BlockSpec((1, _H), lambda i: (0, 0)),
        pl.BlockSpec((_H, _E), lambda i: (0, 0)),
        pl.BlockSpec((1, _E), lambda i: (0, 0)),
        pl.BlockSpec(memory_space=pltpu.SMEM),
    ]
    aliases = {}
    if si > 0:
        in_specs += [pl.BlockSpec(memory_space=pltpu.MemorySpace.HBM)]
        aliases = {6: 0}
    return dict(
        grid=(_NS // _BLK_M,),
        in_specs=in_specs,
        out_specs=[
            pl.BlockSpec((_BLK_M, _E), lambda i: (i + base, 0)),
            pl.BlockSpec((_E, _BLK_M), lambda i: (0, i)),
        ],
        out_shape=[
            jax.ShapeDtypeStruct((_N, _E), jnp.float32),
            jax.ShapeDtypeStruct((_E, _NS), jnp.float32),
        ],
        input_output_aliases=aliases,
    )


def _route_sc_body(gate_t_hbm, rw_t_hbm, se_t_hbm, gt_v, rw_v, se_v):
    wid = lax.axis_index("s") * 2 + lax.axis_index("c")
    base = wid * _CHUNK
    pltpu.sync_copy(gate_t_hbm.at[:, pl.ds(base, _CHUNK)], gt_v)

    def group(g, carry):
        offs = g * _LANES
        vals = [gt_v[e, pl.ds(offs, _LANES)] for e in range(_E)]
        m = _tree(jnp.maximum, vals)
        tv = [jnp.full((_LANES,), _NEG_INF, jnp.float32) for _ in range(_TOPK)]
        ti = [jnp.zeros((_LANES,), jnp.int32) for _ in range(_TOPK)]
        ex = [jnp.exp(v - m) for v in vals]
        s = _tree(jnp.add, ex)
        for e in range(_E):
            v = ex[e]
            iv = jnp.full((_LANES,), e, jnp.int32)
            for j in range(_TOPK):
                gt = v > tv[j]
                nv = jnp.where(gt, v, tv[j])
                ni = jnp.where(gt, iv, ti[j])
                v = jnp.where(gt, tv[j], v)
                iv = jnp.where(gt, ti[j], iv)
                tv[j] = nv
                ti[j] = ni
        r = 1.0 / s
        for j in range(_TOPK):
            rw_v[j, pl.ds(offs, _LANES)] = tv[j] * r
            se_v[j, pl.ds(offs, _LANES)] = ti[j]
        return carry

    lax.fori_loop(0, _GROUPS, group, 0)
    pltpu.sync_copy(rw_v, rw_t_hbm.at[:, pl.ds(base, _CHUNK)])
    pltpu.sync_copy(se_v, se_t_hbm.at[:, pl.ds(base, _CHUNK)])


def _route_sc():
    mesh = plsc.VectorSubcoreMesh(core_axis_name="c", subcore_axis_name="s",
                                  num_cores=2, num_subcores=16)
    return pl.kernel(
        _route_sc_body,
        out_type=[
            jax.ShapeDtypeStruct((_TOPK, _NS), jnp.float32),
            jax.ShapeDtypeStruct((_TOPK, _NS), jnp.int32),
        ],
        mesh=mesh,
        scratch_types=[
            pltpu.VMEM((_E, _CHUNK), jnp.float32),
            pltpu.VMEM((_TOPK, _CHUNK), jnp.float32),
            pltpu.VMEM((_TOPK, _CHUNK), jnp.int32),
        ],
    )


@jax.jit
def kernel(hidden_states, W1, b1, W2, b2, temperature):
    x = hidden_states.reshape(_N, _D)
    b1r, b2r = b1.reshape(1, _H), b2.reshape(1, _E)
    gate = None
    route = _route_sc()
    rws, ses = [], []
    for si in range(_SLICES):
        mlp = pl.pallas_call(_mlp_body, **_mlp_call_kwargs(si))
        args = (x, W1, b1r, W2, b2r, temperature)
        if si > 0:
            args += (gate,)
        gate, gate_t = mlp(*args)
        rw_t, se_t = route(gate_t)
        rws.append(rw_t)
        ses.append(se_t)
    rw_t = lax.concatenate(rws, 1)
    se_t = lax.concatenate(ses, 1)
    return rw_t.T, se_t.T, gate


# speedup vs baseline: 1.0713x; 1.0713x over previous
import jax
import jax.numpy as jnp
from jax import lax
from jax.experimental import pallas as pl
from jax.experimental.pallas import tpu as pltpu
from jax.experimental.pallas import tpu_sc as plsc

_B, _S, _D, _E, _TOPK = 4, 8192, 4096, 64, 8
_H = _D // 4
_N = _B * _S
_BLK_M = 1024

_SLICE_SIZES = (24576, 8192)
_SLICES = len(_SLICE_SIZES)
_NWORKERS = 32
_LANES = 16

_NEG_INF = float("-inf")


def _tree(op, xs):
    xs = list(xs)
    while len(xs) > 1:
        nxt = [op(xs[i], xs[i + 1]) for i in range(0, len(xs) - 1, 2)]
        if len(xs) % 2:
            nxt.append(xs[-1])
        xs = nxt
    return xs[0]


def _mlp_body(x_ref, w1_ref, b1_ref, w2_ref, b2_ref, t_ref, *rest):
    gate_ref, gate_t_ref = rest[-2], rest[-1]
    x = x_ref[...]
    h = jnp.dot(x, w1_ref[...], preferred_element_type=jnp.float32)
    h = jnp.maximum(h + b1_ref[...], 0.0)
    g = jnp.dot(h, w2_ref[...], preferred_element_type=jnp.float32)
    g = (g + b2_ref[...]) * (1.0 / t_ref[0])
    gate_ref[...] = g
    gate_t_ref[...] = g.T


def _mlp_call_kwargs(si):
    ns = _SLICE_SIZES[si]
    base = sum(_SLICE_SIZES[:si]) // _BLK_M
    in_specs = [
        pl.BlockSpec((_BLK_M, _D), lambda i: (i + base, 0)),
        pl.BlockSpec((_D, _H), lambda i: (0, 0)),
        pl.BlockSpec((1, _H), lambda i: (0, 0)),
        pl.BlockSpec((_H, _E), lambda i: (0, 0)),
        pl.BlockSpec((1, _E), lambda i: (0, 0)),
        pl.BlockSpec(memory_space=pltpu.SMEM),
    ]
    aliases = {}
    if si > 0:
        in_specs += [pl.BlockSpec(memory_space=pltpu.MemorySpace.HBM)]
        aliases = {6: 0}
    return dict(
        grid=(ns // _BLK_M,),
        in_specs=in_specs,
        out_specs=[
            pl.BlockSpec((_BLK_M, _E), lambda i: (i + base, 0)),
            pl.BlockSpec((_E, _BLK_M), lambda i: (0, i)),
        ],
        out_shape=[
            jax.ShapeDtypeStruct((_N, _E), jnp.float32),
            jax.ShapeDtypeStruct((_E, ns), jnp.float32),
        ],
        input_output_aliases=aliases,
    )


def _route_sc_body(chunk, gate_t_hbm, rw_t_hbm, se_t_hbm, gt_v, rw_v, se_v):
    groups = chunk // _LANES
    wid = lax.axis_index("s") * 2 + lax.axis_index("c")
    base = wid * chunk
    pltpu.sync_copy(gate_t_hbm.at[:, pl.ds(base, chunk)], gt_v)

    def group(g, carry):
        offs = g * _LANES
        vals = [gt_v[e, pl.ds(offs, _LANES)] for e in range(_E)]
        m = _tree(jnp.maximum, vals)
        tv = [jnp.full((_LANES,), _NEG_INF, jnp.float32) for _ in range(_TOPK)]
        ti = [jnp.zeros((_LANES,), jnp.int32) for _ in range(_TOPK)]
        ex = [jnp.exp(v - m) for v in vals]
        s = _tree(jnp.add, ex)
        for e in range(_E):
            v = ex[e]
            iv = jnp.full((_LANES,), e, jnp.int32)
            for j in range(_TOPK):
                gt = v > tv[j]
                nv = jnp.where(gt, v, tv[j])
                ni = jnp.where(gt, iv, ti[j])
                v = jnp.where(gt, tv[j], v)
                iv = jnp.where(gt, ti[j], iv)
                tv[j] = nv
                ti[j] = ni
        r = 1.0 / s
        for j in range(_TOPK):
            rw_v[j, pl.ds(offs, _LANES)] = tv[j] * r
            se_v[j, pl.ds(offs, _LANES)] = ti[j]
        return carry

    lax.fori_loop(0, groups, group, 0)
    pltpu.sync_copy(rw_v, rw_t_hbm.at[:, pl.ds(base, chunk)])
    pltpu.sync_copy(se_v, se_t_hbm.at[:, pl.ds(base, chunk)])


def _route_sc(ns):
    chunk = ns // _NWORKERS
    mesh = plsc.VectorSubcoreMesh(core_axis_name="c", subcore_axis_name="s",
                                  num_cores=2, num_subcores=16)
    def body(*refs):
        _route_sc_body(chunk, *refs)
    return pl.kernel(
        body,
        out_type=[
            jax.ShapeDtypeStruct((_TOPK, ns), jnp.float32),
            jax.ShapeDtypeStruct((_TOPK, ns), jnp.int32),
        ],
        mesh=mesh,
        scratch_types=[
            pltpu.VMEM((_E, chunk), jnp.float32),
            pltpu.VMEM((_TOPK, chunk), jnp.float32),
            pltpu.VMEM((_TOPK, chunk), jnp.int32),
        ],
    )


@jax.jit
def kernel(hidden_states, W1, b1, W2, b2, temperature):
    x = hidden_states.reshape(_N, _D)
    b1r, b2r = b1.reshape(1, _H), b2.reshape(1, _E)
    gate = None
    rws, ses = [], []
    for si in range(_SLICES):
        mlp = pl.pallas_call(_mlp_body, **_mlp_call_kwargs(si))
        args = (x, W1, b1r, W2, b2r, temperature)
        if si > 0:
            args += (gate,)
        gate, gate_t = mlp(*args)
        rw_t, se_t = _route_sc(_SLICE_SIZES[si])(gate_t)
        rws.append(rw_t)
        ses.append(se_t)
    rw_t = lax.concatenate(rws, 1)
    se_t = lax.concatenate(ses, 1)
    return rw_t.T, se_t.T, gate


# speedup vs baseline: 1.0749x; 1.0033x over previous
import jax
import jax.numpy as jnp
from jax import lax
from jax.experimental import pallas as pl
from jax.experimental.pallas import tpu as pltpu
from jax.experimental.pallas import tpu_sc as plsc

_B, _S, _D, _E, _TOPK = 4, 8192, 4096, 64, 8
_H = _D // 4
_N = _B * _S
_BLK_M = 1024

_SLICE_SIZES = (28672, 4096)
_SLICES = len(_SLICE_SIZES)
_NWORKERS = 32
_LANES = 16

_NEG_INF = float("-inf")


def _tree(op, xs):
    xs = list(xs)
    while len(xs) > 1:
        nxt = [op(xs[i], xs[i + 1]) for i in range(0, len(xs) - 1, 2)]
        if len(xs) % 2:
            nxt.append(xs[-1])
        xs = nxt
    return xs[0]


def _mlp_body(x_ref, w1_ref, b1_ref, w2_ref, b2_ref, t_ref, *rest):
    gate_ref, gate_t_ref = rest[-2], rest[-1]
    x = x_ref[...]
    h = jnp.dot(x, w1_ref[...], preferred_element_type=jnp.float32)
    h = jnp.maximum(h + b1_ref[...], 0.0)
    g = jnp.dot(h, w2_ref[...], preferred_element_type=jnp.float32)
    g = (g + b2_ref[...]) * (1.0 / t_ref[0])
    gate_ref[...] = g
    gate_t_ref[...] = g.T


def _mlp_call_kwargs(si):
    ns = _SLICE_SIZES[si]
    base = sum(_SLICE_SIZES[:si]) // _BLK_M
    in_specs = [
        pl.BlockSpec((_BLK_M, _D), lambda i: (i + base, 0)),
        pl.BlockSpec((_D, _H), lambda i: (0, 0)),
        pl.BlockSpec((1, _H), lambda i: (0, 0)),
        pl.BlockSpec((_H, _E), lambda i: (0, 0)),
        pl.BlockSpec((1, _E), lambda i: (0, 0)),
        pl.BlockSpec(memory_space=pltpu.SMEM),
    ]
    aliases = {}
    if si > 0:
        in_specs += [pl.BlockSpec(memory_space=pltpu.MemorySpace.HBM)]
        aliases = {6: 0}
    return dict(
        grid=(ns // _BLK_M,),
        in_specs=in_specs,
        out_specs=[
            pl.BlockSpec((_BLK_M, _E), lambda i: (i + base, 0)),
            pl.BlockSpec((_E, _BLK_M), lambda i: (0, i)),
        ],
        out_shape=[
            jax.ShapeDtypeStruct((_N, _E), jnp.float32),
            jax.ShapeDtypeStruct((_E, ns), jnp.float32),
        ],
        input_output_aliases=aliases,
    )


def _route_sc_body(chunk, gate_t_hbm, rw_t_hbm, se_t_hbm, gt_v, rw_v, se_v):
    groups = chunk // _LANES
    wid = lax.axis_index("s") * 2 + lax.axis_index("c")
    base = wid * chunk
    pltpu.sync_copy(gate_t_hbm.at[:, pl.ds(base, chunk)], gt_v)

    def group(g, carry):
        offs = g * _LANES
        vals = [gt_v[e, pl.ds(offs, _LANES)] for e in range(_E)]
        m = _tree(jnp.maximum, vals)
        tv = [jnp.full((_LANES,), _NEG_INF, jnp.float32) for _ in range(_TOPK)]
        ti = [jnp.zeros((_LANES,), jnp.int32) for _ in range(_TOPK)]
        ex = [jnp.exp(v - m) for v in vals]
        s = _tree(jnp.add, ex)
        for e in range(_E):
            v = ex[e]
            iv = jnp.full((_LANES,), e, jnp.int32)
            for j in range(_TOPK):
                gt = v > tv[j]
                nv = jnp.where(gt, v, tv[j])
                ni = jnp.where(gt, iv, ti[j])
                v = jnp.where(gt, tv[j], v)
                iv = jnp.where(gt, ti[j], iv)
                tv[j] = nv
                ti[j] = ni
        r = 1.0 / s
        for j in range(_TOPK):
            rw_v[j, pl.ds(offs, _LANES)] = tv[j] * r
            se_v[j, pl.ds(offs, _LANES)] = ti[j]
        return carry

    lax.fori_loop(0, groups, group, 0)
    pltpu.sync_copy(rw_v, rw_t_hbm.at[:, pl.ds(base, chunk)])
    pltpu.sync_copy(se_v, se_t_hbm.at[:, pl.ds(base, chunk)])


def _route_sc(ns):
    chunk = ns // _NWORKERS
    mesh = plsc.VectorSubcoreMesh(core_axis_name="c", subcore_axis_name="s",
                                  num_cores=2, num_subcores=16)
    def body(*refs):
        _route_sc_body(chunk, *refs)
    return pl.kernel(
        body,
        out_type=[
            jax.ShapeDtypeStruct((_TOPK, ns), jnp.float32),
            jax.ShapeDtypeStruct((_TOPK, ns), jnp.int32),
        ],
        mesh=mesh,
        scratch_types=[
            pltpu.VMEM((_E, chunk), jnp.float32),
            pltpu.VMEM((_TOPK, chunk), jnp.float32),
            pltpu.VMEM((_TOPK, chunk), jnp.int32),
        ],
    )


@jax.jit
def kernel(hidden_states, W1, b1, W2, b2, temperature):
    x = hidden_states.reshape(_N, _D)
    b1r, b2r = b1.reshape(1, _H), b2.reshape(1, _E)
    gate = None
    rws, ses = [], []
    for si in range(_SLICES):
        mlp = pl.pallas_call(_mlp_body, **_mlp_call_kwargs(si))
        args = (x, W1, b1r, W2, b2r, temperature)
        if si > 0:
            args += (gate,)
        gate, gate_t = mlp(*args)
        rw_t, se_t = _route_sc(_SLICE_SIZES[si])(gate_t)
        rws.append(rw_t)
        ses.append(se_t)
    rw_t = lax.concatenate(rws, 1)
    se_t = lax.concatenate(ses, 1)
    return rw_t.T, se_t.T, gate
